# baseline (speedup 1.0000x reference)
import functools

import jax
import jax.numpy as jnp
from jax import lax
from jax.experimental import pallas as pl
from jax.experimental.pallas import tpu as pltpu
from jax.experimental.pallas import tpu_sc as plsc


@functools.cache
def _gather_kernel(N, D, NR):
    info = plsc.get_sparse_core_info()
    NC, NS, L = info.num_cores, info.num_subcores, info.num_lanes
    NW = NC * NS
    assert NR % NW == 0 and D % L == 0
    b_per_w = NR // NW
    CH = 16
    while b_per_w % CH:
        CH //= 2
    sched = []
    o = 0
    while o < b_per_w:
        if b_per_w - o == CH and CH >= 16:
            sched.append((o, CH // 2))
            sched.append((o + CH // 2, CH // 2))
            o += CH
        else:
            sched.append((o, CH))
            o += CH
    nch = len(sched)
    NBUF = min(7, nch)
    mesh = plsc.VectorSubcoreMesh(core_axis_name="c", subcore_axis_name="s")

    @functools.partial(
        pl.kernel,
        mesh=mesh,
        out_type=jax.ShapeDtypeStruct((NR, 1, D), jnp.float32),
        scratch_types=[
            pltpu.VMEM((L,), jnp.int32),
            pltpu.VMEM((b_per_w,), jnp.int32),
            pltpu.VMEM((NBUF, CH, D), jnp.float32),
        ]
        + [pltpu.SemaphoreType.DMA] * (2 * NBUF),
    )
    def k(tab_hbm, ids_hbm, off_hbm, out_hbm, off_v, idx_v, ring, *sems):
        rsem, wsem = sems[:NBUF], sems[NBUF:]
        wid = lax.axis_index("s") * NC + lax.axis_index("c")
        base = wid * b_per_w
        pltpu.sync_copy(ids_hbm.at[pl.ds(base, b_per_w)], idx_v)
        pltpu.sync_copy(off_hbm, off_v)
        ov = off_v[...]
        for i in range(b_per_w // L):
            idx_v[pl.ds(i * L, L)] = idx_v[pl.ds(i * L, L)] + ov

        def start_gather(c):
            o, sz = sched[c]
            return pltpu.async_copy(
                tab_hbm.at[idx_v.at[pl.ds(o, sz)]],
                ring.at[c % NBUF, pl.ds(0, sz)], rsem[c % NBUF])

        def start_write(c):
            o, sz = sched[c]
            return pltpu.async_copy(
                ring.at[c % NBUF, pl.ds(0, sz)],
                out_hbm.at[pl.ds(base + o, sz), 0], wsem[c % NBUF])

        rh = [None] * NBUF
        wh = [None] * NBUF
        for c in range(NBUF):
            rh[c] = start_gather(c)
        for c in range(nch):
            rh[c % NBUF].wait()
            wh[c % NBUF] = start_write(c)
            if c + NBUF < nch:
                wh[c % NBUF].wait()
                rh[c % NBUF] = start_gather(c + NBUF)
        for c in range(nch):
            if c + NBUF >= nch:
                wh[c % NBUF].wait()

    return k


def kernel(tables, token_ids, group_idx):
    G, V, D = tables.shape
    B, S = token_ids.shape
    flat_tab = tables.reshape(G * V, D)
    ids = token_ids.reshape(B * S)
    info = plsc.get_sparse_core_info()
    off = jnp.broadcast_to(
        jnp.asarray(group_idx, jnp.int32) * jnp.int32(V), (info.num_lanes,))
    out = _gather_kernel(G * V, D, B * S)(flat_tab, ids, off)
    return out.reshape(B, S, D)


# speedup vs baseline: 1.0037x; 1.0037x over previous
import functools

import jax
import jax.numpy as jnp
from jax import lax
from jax.experimental import pallas as pl
from jax.experimental.pallas import tpu as pltpu
from jax.experimental.pallas import tpu_sc as plsc


@functools.cache
def _gather_kernel(N, D, NR):
    info = plsc.get_sparse_core_info()
    NC, NS, L = info.num_cores, info.num_subcores, info.num_lanes
    NW = NC * NS
    assert NR % NW == 0 and D % L == 0
    b_per_w = NR // NW
    CH = 16
    while b_per_w % CH:
        CH //= 2
    sched = [(o, CH) for o in range(0, b_per_w, CH)]
    nch = len(sched)
    NBUF = min(6, nch)
    mesh = plsc.VectorSubcoreMesh(core_axis_name="c", subcore_axis_name="s")

    @functools.partial(
        pl.kernel,
        mesh=mesh,
        out_type=jax.ShapeDtypeStruct((NR, 1, D), jnp.float32),
        scratch_types=[
            pltpu.VMEM((L,), jnp.int32),
            pltpu.VMEM((b_per_w,), jnp.int32),
            pltpu.VMEM((NBUF, CH, D), jnp.float32),
        ]
        + [pltpu.SemaphoreType.DMA] * (2 * NBUF),
    )
    def k(tab_hbm, ids_hbm, off_hbm, out_hbm, off_v, idx_v, ring, *sems):
        rsem, wsem = sems[:NBUF], sems[NBUF:]
        wid = lax.axis_index("s") * NC + lax.axis_index("c")
        base = wid * b_per_w
        pltpu.sync_copy(ids_hbm.at[pl.ds(base, b_per_w)], idx_v)
        pltpu.sync_copy(off_hbm, off_v)
        ov = off_v[...]
        for i in range(b_per_w // L):
            idx_v[pl.ds(i * L, L)] = idx_v[pl.ds(i * L, L)] + ov

        def start_gather(c):
            o, sz = sched[c]
            return pltpu.async_copy(
                tab_hbm.at[idx_v.at[pl.ds(o, sz)]],
                ring.at[c % NBUF, pl.ds(0, sz)], rsem[c % NBUF])

        def start_write(c):
            o, sz = sched[c]
            return pltpu.async_copy(
                ring.at[c % NBUF, pl.ds(0, sz)],
                out_hbm.at[pl.ds(base + o, sz), 0], wsem[c % NBUF])

        rh = [None] * NBUF
        wh = [None] * NBUF
        for c in range(NBUF):
            rh[c] = start_gather(c)
        for c in range(nch):
            rh[c % NBUF].wait()
            wh[c % NBUF] = start_write(c)
            if c + NBUF < nch:
                wh[c % NBUF].wait()
                rh[c % NBUF] = start_gather(c + NBUF)
        for c in range(nch):
            if c + NBUF >= nch:
                wh[c % NBUF].wait()

    return k


def kernel(tables, token_ids, group_idx):
    G, V, D = tables.shape
    B, S = token_ids.shape
    flat_tab = tables.reshape(G * V, D)
    ids = token_ids.reshape(B * S)
    info = plsc.get_sparse_core_info()
    off = jnp.broadcast_to(
        jnp.asarray(group_idx, jnp.int32) * jnp.int32(V), (info.num_lanes,))
    out = _gather_kernel(G * V, D, B * S)(flat_tab, ids, off)
    return out.reshape(B, S, D)


# speedup vs baseline: 1.0171x; 1.0133x over previous
import functools

import jax
import jax.numpy as jnp
from jax import lax
from jax.experimental import pallas as pl
from jax.experimental.pallas import tpu as pltpu
from jax.experimental.pallas import tpu_sc as plsc


@functools.cache
def _gather_kernel(N, D, NR, V):
    info = plsc.get_sparse_core_info()
    NC, NS, L = info.num_cores, info.num_subcores, info.num_lanes
    NW = NC * NS
    assert NR % NW == 0 and D % L == 0
    b_per_w = NR // NW
    CH = 16
    while b_per_w % CH:
        CH //= 2
    sched = [(o, CH) for o in range(0, b_per_w, CH)]
    nch = len(sched)
    NBUF = min(6, nch)
    mesh = plsc.VectorSubcoreMesh(core_axis_name="c", subcore_axis_name="s")

    @functools.partial(
        pl.kernel,
        mesh=mesh,
        out_type=jax.ShapeDtypeStruct((NR, 1, D), jnp.float32),
        scratch_types=[
            pltpu.VMEM((L,), jnp.int32),
            pltpu.VMEM((b_per_w,), jnp.int32),
            pltpu.VMEM((NBUF, CH, D), jnp.float32),
        ]
        + [pltpu.SemaphoreType.DMA] * (2 * NBUF),
    )
    def k(tab_hbm, ids_hbm, off_hbm, out_hbm, off_v, idx_v, ring, *sems):
        rsem, wsem = sems[:NBUF], sems[NBUF:]
        wid = lax.axis_index("s") * NC + lax.axis_index("c")
        base = wid * b_per_w
        idsh = pltpu.async_copy(
            ids_hbm.at[pl.ds(base, b_per_w)], idx_v, rsem[0])
        pltpu.sync_copy(off_hbm, off_v)
        ov = off_v[...]
        idsh.wait()
        for i in range(b_per_w // L):
            idx_v[pl.ds(i * L, L)] = idx_v[pl.ds(i * L, L)] + ov

        def start_gather(c):
            o, sz = sched[c]
            return pltpu.async_copy(
                tab_hbm.at[idx_v.at[pl.ds(o, sz)]],
                ring.at[c % NBUF, pl.ds(0, sz)], rsem[c % NBUF])

        def start_write(c):
            o, sz = sched[c]
            return pltpu.async_copy(
                ring.at[c % NBUF, pl.ds(0, sz)],
                out_hbm.at[pl.ds(base + o, sz), 0], wsem[c % NBUF])

        rh = [None] * NBUF
        wh = [None] * NBUF
        for c in range(NBUF):
            rh[c] = start_gather(c)
        for c in range(nch):
            rh[c % NBUF].wait()
            wh[c % NBUF] = start_write(c)
            if c + NBUF < nch:
                wh[c % NBUF].wait()
                rh[c % NBUF] = start_gather(c + NBUF)
        for c in range(nch):
            if c + NBUF >= nch:
                wh[c % NBUF].wait()

    return k


def kernel(tables, token_ids, group_idx):
    G, V, D = tables.shape
    B, S = token_ids.shape
    flat_tab = tables.reshape(G * V, D)
    ids = token_ids.reshape(B * S)
    off = jnp.broadcast_to(
        jnp.asarray(group_idx, jnp.int32) * jnp.int32(V), (16,))
    out = _gather_kernel(G * V, D, B * S, V)(flat_tab, ids, off)
    return out.reshape(B, S, D)
